# Initial kernel scaffold; baseline (speedup 1.0000x reference)
import jax
import jax.numpy as jnp
from jax.experimental import pallas as pl


def kernel(inputs, shared_embed_weight, pos_emb_cache, modality_embedding):
    raise NotImplementedError("write your pallas kernel here")


# speedup vs baseline: 1.2465x; 1.2465x over previous
import functools

import jax
import jax.numpy as jnp
from jax import lax
from jax.experimental import pallas as pl
from jax.experimental.pallas import tpu as pltpu
from jax.experimental.pallas import tpu_sc as plsc

_NC = 2
_NS = 16
_NW = _NC * _NS
_L = 16


def _sc_embed_gather(table, idx_flat, modality):
  V, D = table.shape
  B = idx_flat.shape[0]
  b_per_w = B // _NW
  R = 16
  n_chunks = b_per_w // R
  mesh = plsc.VectorSubcoreMesh(core_axis_name="c", subcore_axis_name="s")

  @functools.partial(
      pl.kernel,
      mesh=mesh,
      out_type=jax.ShapeDtypeStruct((B, D), jnp.float32),
      scratch_types=[
          pltpu.VMEM((b_per_w,), jnp.int32),
          pltpu.VMEM((R, D), jnp.float32),
          pltpu.VMEM((D,), jnp.float32),
          pltpu.SemaphoreType.DMA,
      ],
  )
  def k(table_hbm, idx_hbm, mod_hbm, out_hbm, idx_v, rows_v, mod_v, sem):
    wid = lax.axis_index("s") * _NC + lax.axis_index("c")
    base = wid * b_per_w
    pltpu.sync_copy(idx_hbm.at[pl.ds(base, b_per_w)], idx_v)
    pltpu.sync_copy(mod_hbm, mod_v)
    for c in range(n_chunks):
      pltpu.async_copy(
          table_hbm.at[idx_v.at[pl.ds(c * R, R)]], rows_v, sem).wait()

      def row_body(r, _):
        def col_body(j, _):
          col = j * _L
          rows_v[r, pl.ds(col, _L)] = (
              rows_v[r, pl.ds(col, _L)] + mod_v[pl.ds(col, _L)])
          return 0
        return lax.fori_loop(0, D // _L, col_body, 0)

      lax.fori_loop(0, R, row_body, 0)
      pltpu.sync_copy(rows_v, out_hbm.at[pl.ds(base + c * R, R)])

  return k(table, idx_flat, modality)


def _tc_pos_emb(cache, bs):
  S, D = cache.shape
  blk = 256

  def body(c_ref, o_ref):
    o_ref[...] = c_ref[...][None]

  return pl.pallas_call(
      body,
      grid=(bs, S // blk),
      in_specs=[pl.BlockSpec((blk, D), lambda b, i: (i, 0))],
      out_specs=pl.BlockSpec((1, blk, D), lambda b, i: (b, i, 0)),
      out_shape=jax.ShapeDtypeStruct((bs, S, D), jnp.float32),
  )(cache)


def _tc_ones(rows, S):
  blk = 512

  def body(o_ref):
    o_ref[...] = jnp.ones_like(o_ref)

  return pl.pallas_call(
      body,
      grid=(rows, S // blk),
      out_specs=pl.BlockSpec((1, blk, S), lambda i, j: (i, j, 0)),
      out_shape=jax.ShapeDtypeStruct((rows, S, S), jnp.float32),
  )()


def kernel(inputs, shared_embed_weight, pos_emb_cache, modality_embedding):
  bs, seq_len = inputs.shape
  emb_dim = shared_embed_weight.shape[1]

  x = _sc_embed_gather(
      shared_embed_weight, inputs.reshape(-1), modality_embedding)
  x = x.reshape(bs, seq_len, emb_dim)

  pos_emb = _tc_pos_emb(pos_emb_cache, bs)

  attn_pattern_mask = _tc_ones(bs * 4, seq_len).reshape(
      bs, 4, seq_len, seq_len)

  modality_id = jnp.array(0, dtype=jnp.int32)
  return (x, pos_emb, modality_id, attn_pattern_mask)


# speedup vs baseline: 1.8657x; 1.4968x over previous
import functools

import jax
import jax.numpy as jnp
from jax import lax
from jax.experimental import pallas as pl
from jax.experimental.pallas import tpu as pltpu
from jax.experimental.pallas import tpu_sc as plsc

_NC = 2
_NS = 16
_NW = _NC * _NS
_L = 16


def _sc_embed_gather(table, idx_flat, modality):
  V, D = table.shape
  B = idx_flat.shape[0]
  b_per_w = B // _NW
  R = 8
  NBUF = 6
  n_chunks = b_per_w // R
  mesh = plsc.VectorSubcoreMesh(core_axis_name="c", subcore_axis_name="s")

  @functools.partial(
      pl.kernel,
      mesh=mesh,
      out_type=jax.ShapeDtypeStruct((B, D), jnp.float32),
      scratch_types=[
          pltpu.VMEM((b_per_w,), jnp.int32),
          pltpu.VMEM((NBUF, R, D), jnp.float32),
          pltpu.VMEM((D,), jnp.float32),
          pltpu.SemaphoreType.DMA((NBUF,)),
          pltpu.SemaphoreType.DMA((NBUF,)),
      ],
  )
  def k(table_hbm, idx_hbm, mod_hbm, out_hbm, idx_v, rows_v, mod_v,
        gsem, osem):
    wid = lax.axis_index("s") * _NC + lax.axis_index("c")
    base = wid * b_per_w
    pltpu.sync_copy(idx_hbm.at[pl.ds(base, b_per_w)], idx_v)
    pltpu.sync_copy(mod_hbm, mod_v)

    def start_gather(c):
      b = c % NBUF
      return pltpu.async_copy(
          table_hbm.at[idx_v.at[pl.ds(c * R, R)]], rows_v.at[b],
          gsem.at[b])

    g = {c: start_gather(c) for c in range(min(NBUF - 1, n_chunks))}
    o = {}
    for c in range(n_chunks):
      b = c % NBUF
      g[c].wait()
      nxt = c + NBUF - 1
      if nxt < n_chunks:
        prev = nxt - NBUF
        if prev >= 0:
          o[prev].wait()
        g[nxt] = start_gather(nxt)

      def col_body(j, _):
        m = mod_v[pl.ds(j * _L, _L)]
        for r in range(R):
          rows_v[b, r, pl.ds(j * _L, _L)] = (
              rows_v[b, r, pl.ds(j * _L, _L)] + m)
        return 0

      lax.fori_loop(0, D // _L, col_body, 0)
      o[c] = pltpu.async_copy(
          rows_v.at[b], out_hbm.at[pl.ds(base + c * R, R)], osem.at[b])
    for c in range(max(0, n_chunks - NBUF), n_chunks):
      o[c].wait()

  return k(table, idx_flat, modality)


def _tc_pos_emb(cache, bs):
  S, D = cache.shape
  blk = 256

  def body(c_ref, o_ref):
    o_ref[...] = c_ref[...][None]

  return pl.pallas_call(
      body,
      grid=(bs, S // blk),
      in_specs=[pl.BlockSpec((blk, D), lambda b, i: (i, 0))],
      out_specs=pl.BlockSpec((1, blk, D), lambda b, i: (b, i, 0)),
      out_shape=jax.ShapeDtypeStruct((bs, S, D), jnp.float32),
  )(cache)


def _tc_ones(rows, S):
  blk = 512

  def body(o_ref):
    o_ref[...] = jnp.ones_like(o_ref)

  return pl.pallas_call(
      body,
      grid=(rows, S // blk),
      out_specs=pl.BlockSpec((1, blk, S), lambda i, j: (i, j, 0)),
      out_shape=jax.ShapeDtypeStruct((rows, S, S), jnp.float32),
  )()


def kernel(inputs, shared_embed_weight, pos_emb_cache, modality_embedding):
  bs, seq_len = inputs.shape
  emb_dim = shared_embed_weight.shape[1]

  x = _sc_embed_gather(
      shared_embed_weight, inputs.reshape(-1), modality_embedding)
  x = x.reshape(bs, seq_len, emb_dim)

  pos_emb = _tc_pos_emb(pos_emb_cache, bs)

  attn_pattern_mask = _tc_ones(bs * 4, seq_len).reshape(
      bs, 4, seq_len, seq_len)

  modality_id = jnp.array(0, dtype=jnp.int32)
  return (x, pos_emb, modality_id, attn_pattern_mask)


# speedup vs baseline: 1.9584x; 1.0497x over previous
import functools

import jax
import jax.numpy as jnp
from jax import lax
from jax.experimental import pallas as pl
from jax.experimental.pallas import tpu as pltpu
from jax.experimental.pallas import tpu_sc as plsc

_NC = 2
_NS = 16
_NW = _NC * _NS
_L = 16


def _sc_embed_gather(table, idx_flat, modality, pos_cache, bs):
  V, D = table.shape
  B = idx_flat.shape[0]
  S = pos_cache.shape[0]
  b_per_w = B // _NW
  R = 8
  NBUF = 6
  n_chunks = b_per_w // R
  p_per_w = S // _NW
  n_pchunks = p_per_w // R
  mesh = plsc.VectorSubcoreMesh(core_axis_name="c", subcore_axis_name="s")

  @functools.partial(
      pl.kernel,
      mesh=mesh,
      out_type=(
          jax.ShapeDtypeStruct((B, D), jnp.float32),
          jax.ShapeDtypeStruct((bs * S, D), jnp.float32),
      ),
      scratch_types=[
          pltpu.VMEM((b_per_w,), jnp.int32),
          pltpu.VMEM((NBUF, R, D), jnp.float32),
          pltpu.VMEM((D,), jnp.float32),
          pltpu.SemaphoreType.DMA((NBUF,)),
          pltpu.SemaphoreType.DMA((NBUF,)),
      ],
  )
  def k(table_hbm, idx_hbm, mod_hbm, pos_hbm, out_hbm, pout_hbm,
        idx_v, rows_v, mod_v, gsem, osem):
    wid = lax.axis_index("s") * _NC + lax.axis_index("c")
    base = wid * b_per_w
    pltpu.sync_copy(idx_hbm.at[pl.ds(base, b_per_w)], idx_v)
    pltpu.sync_copy(mod_hbm, mod_v)

    def start_gather(c):
      b = c % NBUF
      return pltpu.async_copy(
          table_hbm.at[idx_v.at[pl.ds(c * R, R)]], rows_v.at[b],
          gsem.at[b])

    g = {c: start_gather(c) for c in range(min(NBUF - 1, n_chunks))}
    o = {}
    for c in range(n_chunks):
      b = c % NBUF
      g[c].wait()
      nxt = c + NBUF - 1
      if nxt < n_chunks:
        prev = nxt - NBUF
        if prev >= 0:
          o[prev].wait()
        g[nxt] = start_gather(nxt)

      def col_body(j, _):
        m = mod_v[pl.ds(j * _L, _L)]
        for r in range(R):
          rows_v[b, r, pl.ds(j * _L, _L)] = (
              rows_v[b, r, pl.ds(j * _L, _L)] + m)
        return 0

      lax.fori_loop(0, D // _L, col_body, 0)

      o[c] = pltpu.async_copy(
          rows_v.at[b], out_hbm.at[pl.ds(base + c * R, R)], osem.at[b])
    for c in range(max(0, n_chunks - NBUF), n_chunks):
      o[c].wait()

    pbase = wid * p_per_w

    def start_pin(c):
      b = c % 3
      return pltpu.async_copy(
          pos_hbm.at[pl.ds(pbase + c * R, R)], rows_v.at[b], gsem.at[b])

    pin = {c: start_pin(c) for c in range(min(2, n_pchunks))}
    po = {}
    for c in range(n_pchunks):
      b = c % 3
      pin[c].wait()
      nxt = c + 2
      if nxt < n_pchunks:
        prev = nxt - 3
        if prev >= 0:
          for q in range(bs):
            po[(prev, q)].wait()
        pin[nxt] = start_pin(nxt)
      for q in range(bs):
        po[(c, q)] = pltpu.async_copy(
            rows_v.at[b],
            pout_hbm.at[pl.ds(q * S + pbase + c * R, R)], osem.at[b])
    for c in range(max(0, n_pchunks - 3), n_pchunks):
      for q in range(bs):
        po[(c, q)].wait()

  return k(table, idx_flat, modality, pos_cache)


def _tc_ones(rows, S):
  blk = 512

  def body(o_ref):
    o_ref[...] = jnp.ones_like(o_ref)

  return pl.pallas_call(
      body,
      grid=(rows, S // blk),
      out_specs=pl.BlockSpec((1, blk, S), lambda i, j: (i, j, 0)),
      out_shape=jax.ShapeDtypeStruct((rows, S, S), jnp.float32),
  )()


def kernel(inputs, shared_embed_weight, pos_emb_cache, modality_embedding):
  bs, seq_len = inputs.shape
  emb_dim = shared_embed_weight.shape[1]

  x, pos_emb = _sc_embed_gather(
      shared_embed_weight, inputs.reshape(-1), modality_embedding,
      pos_emb_cache, bs)
  x = x.reshape(bs, seq_len, emb_dim)
  pos_emb = pos_emb.reshape(bs, seq_len, emb_dim)

  attn_pattern_mask = _tc_ones(bs * 4, seq_len).reshape(
      bs, 4, seq_len, seq_len)

  modality_id = jnp.array(0, dtype=jnp.int32)
  return (x, pos_emb, modality_id, attn_pattern_mask)
